# baseline (speedup 1.0000x reference)
import jax
import jax.numpy as jnp
from jax.experimental import pallas as pl
from jax.experimental.pallas import tpu as pltpu

_B = 32
_ROWS = _B * 3


def _params_body(cam_s, id_s, wcam_ref, bcam_ref, wident_ref, bident_ref,
                 wt_any, bt_any, wv_ref, bv_ref, wscr, bscr, sem):
    cps = []
    for i in range(_B):
        c8 = (cam_s[i] // 8) * 8
        l0 = (id_s[i] // 128) * 128
        cps.append(pltpu.make_async_copy(
            wt_any.at[:, pl.ds(c8, 8), pl.ds(l0, 128)], wscr.at[i], sem))
        cps.append(pltpu.make_async_copy(
            bt_any.at[:, pl.ds(c8, 8), pl.ds(l0, 128)], bscr.at[i], sem))
    for cp in cps:
        cp.start()
    for cp in cps:
        cp.wait()
    sub_i = jax.lax.broadcasted_iota(jnp.int32, (8, 128), 0)
    lane_i = jax.lax.broadcasted_iota(jnp.int32, (8, 128), 1)
    for i in range(_B):
        ci = cam_s[i]
        ii = id_s[i]
        msk = jnp.where((sub_i == ci % 8) & (lane_i == ii % 128), 1.0, 0.0)
        w3 = jnp.sum(wscr[i] * msk[None, :, :], axis=(1, 2))
        b3 = jnp.sum(bscr[i] * msk[None, :, :], axis=(1, 2))
        wrow = (wcam_ref[pl.ds(ci, 1), :] + wident_ref[pl.ds(ii, 1), :]
                + 10.0 * w3.reshape(1, 3))
        brow = (bcam_ref[pl.ds(ci, 1), :] + bident_ref[pl.ds(ii, 1), :]
                + 10.0 * b3.reshape(1, 3))
        wv_ref[pl.ds(i, 1), pl.ds(0, 3)] = wrow
        bv_ref[pl.ds(i, 1), pl.ds(0, 3)] = brow


def _params(cam, idn, wcam, bcam, wident, bident, wt, bt):
    grid_spec = pltpu.PrefetchScalarGridSpec(
        num_scalar_prefetch=2,
        grid=(1,),
        in_specs=[
            pl.BlockSpec((100, 3), lambda i, cs, ids: (0, 0)),
            pl.BlockSpec((100, 3), lambda i, cs, ids: (0, 0)),
            pl.BlockSpec((5000, 3), lambda i, cs, ids: (0, 0)),
            pl.BlockSpec((5000, 3), lambda i, cs, ids: (0, 0)),
            pl.BlockSpec(memory_space=pl.ANY),
            pl.BlockSpec(memory_space=pl.ANY),
        ],
        out_specs=[
            pl.BlockSpec((_B, 8), lambda i, cs, ids: (0, 0)),
            pl.BlockSpec((_B, 8), lambda i, cs, ids: (0, 0)),
        ],
        scratch_shapes=[
            pltpu.VMEM((_B, 3, 8, 128), jnp.float32),
            pltpu.VMEM((_B, 3, 8, 128), jnp.float32),
            pltpu.SemaphoreType.DMA,
        ],
    )
    return pl.pallas_call(
        _params_body,
        grid_spec=grid_spec,
        out_shape=[jax.ShapeDtypeStruct((_B, 8), jnp.float32)] * 2,
    )(cam, idn, wcam, bcam, wident, bident, wt, bt)


_G = 4


def _scale_body(wv_ref, bv_ref, img_ref, out_ref):
    i = pl.program_id(0)
    for j in range(_G):
        r = i * _G + j
        b = r // 3
        c = r - 3 * b
        out_ref[j] = img_ref[j] * wv_ref[b, c] + bv_ref[b, c]


def _scale(wv, bv, img, h, ww):
    smem = pl.BlockSpec(memory_space=pltpu.SMEM)
    return pl.pallas_call(
        _scale_body,
        grid=(_ROWS // _G,),
        in_specs=[smem, smem, pl.BlockSpec((_G, h, ww), lambda i: (i, 0, 0))],
        out_specs=pl.BlockSpec((_G, h, ww), lambda i: (i, 0, 0)),
        out_shape=jax.ShapeDtypeStruct((_ROWS, h, ww), jnp.float32),
    )(wv, bv, img)


def kernel(image, camindex, idindex, wcam, bcam, wident, bident, w, b):
    bsz, ch, h, ww = image.shape
    cam = camindex.astype(jnp.int32)
    idn = idindex.astype(jnp.int32)
    wt = jnp.transpose(w, (2, 0, 1))
    bt = jnp.transpose(b, (2, 0, 1))
    wv, bv = _params(cam, idn, wcam, bcam, wident, bident, wt, bt)
    out = _scale(wv, bv, image.reshape(bsz * ch, h, ww), h, ww)
    return out.reshape(bsz, ch, h, ww)


# speedup vs baseline: 1.0233x; 1.0233x over previous
import jax
import jax.numpy as jnp
from jax.experimental import pallas as pl
from jax.experimental.pallas import tpu as pltpu

_B = 32
_ROWS = _B * 3


def _params_body(cam_s, id_s, wcam_ref, bcam_ref, wident_ref, bident_ref,
                 wt_any, bt_any, wv_ref, bv_ref, wscr, bscr, sem):
    cps = []
    for i in range(_B):
        c8 = (cam_s[i] // 8) * 8
        l0 = (id_s[i] // 128) * 128
        cps.append(pltpu.make_async_copy(
            wt_any.at[:, pl.ds(c8, 8), pl.ds(l0, 128)], wscr.at[i], sem))
        cps.append(pltpu.make_async_copy(
            bt_any.at[:, pl.ds(c8, 8), pl.ds(l0, 128)], bscr.at[i], sem))
    for cp in cps:
        cp.start()
    for cp in cps:
        cp.wait()
    sub_i = jax.lax.broadcasted_iota(jnp.int32, (8, 128), 0)
    lane_i = jax.lax.broadcasted_iota(jnp.int32, (8, 128), 1)
    for i in range(_B):
        ci = cam_s[i]
        ii = id_s[i]
        msk = jnp.where((sub_i == ci % 8) & (lane_i == ii % 128), 1.0, 0.0)
        w3 = jnp.sum(wscr[i] * msk[None, :, :], axis=(1, 2))
        b3 = jnp.sum(bscr[i] * msk[None, :, :], axis=(1, 2))
        wrow = (wcam_ref[pl.ds(ci, 1), :] + wident_ref[pl.ds(ii, 1), :]
                + 10.0 * w3.reshape(1, 3))
        brow = (bcam_ref[pl.ds(ci, 1), :] + bident_ref[pl.ds(ii, 1), :]
                + 10.0 * b3.reshape(1, 3))
        wv_ref[pl.ds(i, 1), pl.ds(0, 3)] = wrow
        bv_ref[pl.ds(i, 1), pl.ds(0, 3)] = brow


def _params(cam, idn, wcam, bcam, wident, bident, wt, bt):
    grid_spec = pltpu.PrefetchScalarGridSpec(
        num_scalar_prefetch=2,
        grid=(1,),
        in_specs=[
            pl.BlockSpec((100, 3), lambda i, cs, ids: (0, 0)),
            pl.BlockSpec((100, 3), lambda i, cs, ids: (0, 0)),
            pl.BlockSpec((5000, 3), lambda i, cs, ids: (0, 0)),
            pl.BlockSpec((5000, 3), lambda i, cs, ids: (0, 0)),
            pl.BlockSpec(memory_space=pl.ANY),
            pl.BlockSpec(memory_space=pl.ANY),
        ],
        out_specs=[
            pl.BlockSpec((_B, 8), lambda i, cs, ids: (0, 0)),
            pl.BlockSpec((_B, 8), lambda i, cs, ids: (0, 0)),
        ],
        scratch_shapes=[
            pltpu.VMEM((_B, 3, 8, 128), jnp.float32),
            pltpu.VMEM((_B, 3, 8, 128), jnp.float32),
            pltpu.SemaphoreType.DMA,
        ],
    )
    return pl.pallas_call(
        _params_body,
        grid_spec=grid_spec,
        out_shape=[jax.ShapeDtypeStruct((_B, 8), jnp.float32)] * 2,
    )(cam, idn, wcam, bcam, wident, bident, wt, bt)


_G = 8


def _scale_body(wv_ref, bv_ref, img_ref, out_ref):
    i = pl.program_id(0)
    for j in range(_G):
        r = i * _G + j
        b = r // 3
        c = r - 3 * b
        out_ref[j] = img_ref[j] * wv_ref[b, c] + bv_ref[b, c]


def _scale(wv, bv, img, h, ww):
    smem = pl.BlockSpec(memory_space=pltpu.SMEM)
    return pl.pallas_call(
        _scale_body,
        grid=(_ROWS // _G,),
        in_specs=[smem, smem, pl.BlockSpec((_G, h, ww), lambda i: (i, 0, 0))],
        out_specs=pl.BlockSpec((_G, h, ww), lambda i: (i, 0, 0)),
        out_shape=jax.ShapeDtypeStruct((_ROWS, h, ww), jnp.float32),
    )(wv, bv, img)


def kernel(image, camindex, idindex, wcam, bcam, wident, bident, w, b):
    bsz, ch, h, ww = image.shape
    cam = camindex.astype(jnp.int32)
    idn = idindex.astype(jnp.int32)
    wt = jnp.transpose(w, (2, 0, 1))
    bt = jnp.transpose(b, (2, 0, 1))
    wv, bv = _params(cam, idn, wcam, bcam, wident, bident, wt, bt)
    out = _scale(wv, bv, image.reshape(bsz * ch, h, ww), h, ww)
    return out.reshape(bsz, ch, h, ww)


# speedup vs baseline: 1.0269x; 1.0035x over previous
import jax
import jax.numpy as jnp
from jax.experimental import pallas as pl
from jax.experimental.pallas import tpu as pltpu

_B = 32
_ROWS = _B * 3


def _params_body(cam_s, id_s, wcam_ref, bcam_ref, wident_ref, bident_ref,
                 wt_any, bt_any, wv_ref, bv_ref, wscr, bscr, sem):
    cps = []
    for i in range(_B):
        c8 = (cam_s[i] // 8) * 8
        l0 = (id_s[i] // 128) * 128
        cps.append(pltpu.make_async_copy(
            wt_any.at[:, pl.ds(c8, 8), pl.ds(l0, 128)], wscr.at[i], sem))
        cps.append(pltpu.make_async_copy(
            bt_any.at[:, pl.ds(c8, 8), pl.ds(l0, 128)], bscr.at[i], sem))
    for cp in cps:
        cp.start()
    for cp in cps:
        cp.wait()
    sub_i = jax.lax.broadcasted_iota(jnp.int32, (8, 128), 0)
    lane_i = jax.lax.broadcasted_iota(jnp.int32, (8, 128), 1)
    for i in range(_B):
        ci = cam_s[i]
        ii = id_s[i]
        msk = jnp.where((sub_i == ci % 8) & (lane_i == ii % 128), 1.0, 0.0)
        w3 = jnp.sum(wscr[i] * msk[None, :, :], axis=(1, 2))
        b3 = jnp.sum(bscr[i] * msk[None, :, :], axis=(1, 2))
        wrow = (wcam_ref[pl.ds(ci, 1), :] + wident_ref[pl.ds(ii, 1), :]
                + 10.0 * w3.reshape(1, 3))
        brow = (bcam_ref[pl.ds(ci, 1), :] + bident_ref[pl.ds(ii, 1), :]
                + 10.0 * b3.reshape(1, 3))
        wv_ref[pl.ds(i, 1), pl.ds(0, 3)] = wrow
        bv_ref[pl.ds(i, 1), pl.ds(0, 3)] = brow


def _params(cam, idn, wcam, bcam, wident, bident, wt, bt):
    grid_spec = pltpu.PrefetchScalarGridSpec(
        num_scalar_prefetch=2,
        grid=(1,),
        in_specs=[
            pl.BlockSpec((100, 3), lambda i, cs, ids: (0, 0)),
            pl.BlockSpec((100, 3), lambda i, cs, ids: (0, 0)),
            pl.BlockSpec((5000, 3), lambda i, cs, ids: (0, 0)),
            pl.BlockSpec((5000, 3), lambda i, cs, ids: (0, 0)),
            pl.BlockSpec(memory_space=pl.ANY),
            pl.BlockSpec(memory_space=pl.ANY),
        ],
        out_specs=[
            pl.BlockSpec((_B, 8), lambda i, cs, ids: (0, 0)),
            pl.BlockSpec((_B, 8), lambda i, cs, ids: (0, 0)),
        ],
        scratch_shapes=[
            pltpu.VMEM((_B, 3, 8, 128), jnp.float32),
            pltpu.VMEM((_B, 3, 8, 128), jnp.float32),
            pltpu.SemaphoreType.DMA,
        ],
    )
    return pl.pallas_call(
        _params_body,
        grid_spec=grid_spec,
        out_shape=[jax.ShapeDtypeStruct((_B, 8), jnp.float32)] * 2,
    )(cam, idn, wcam, bcam, wident, bident, wt, bt)


_G = 12


def _scale_body(wv_ref, bv_ref, img_ref, out_ref):
    i = pl.program_id(0)
    for j in range(_G):
        r = i * _G + j
        b = r // 3
        c = r - 3 * b
        out_ref[j] = img_ref[j] * wv_ref[b, c] + bv_ref[b, c]


def _scale(wv, bv, img, h, ww):
    smem = pl.BlockSpec(memory_space=pltpu.SMEM)
    return pl.pallas_call(
        _scale_body,
        grid=(_ROWS // _G,),
        in_specs=[smem, smem, pl.BlockSpec((_G, h, ww), lambda i: (i, 0, 0))],
        out_specs=pl.BlockSpec((_G, h, ww), lambda i: (i, 0, 0)),
        out_shape=jax.ShapeDtypeStruct((_ROWS, h, ww), jnp.float32),
    )(wv, bv, img)


def kernel(image, camindex, idindex, wcam, bcam, wident, bident, w, b):
    bsz, ch, h, ww = image.shape
    cam = camindex.astype(jnp.int32)
    idn = idindex.astype(jnp.int32)
    wt = jnp.transpose(w, (2, 0, 1))
    bt = jnp.transpose(b, (2, 0, 1))
    wv, bv = _params(cam, idn, wcam, bcam, wident, bident, wt, bt)
    out = _scale(wv, bv, image.reshape(bsz * ch, h, ww), h, ww)
    return out.reshape(bsz, ch, h, ww)


# speedup vs baseline: 1.0365x; 1.0094x over previous
import jax
import jax.numpy as jnp
from jax.experimental import pallas as pl
from jax.experimental.pallas import tpu as pltpu

_B = 32
_ROWS = _B * 3
_G = 12


def _body(cam_s, id_s, wcam_ref, bcam_ref, wident_ref, bident_ref,
          wt_any, bt_any, img_ref, out_ref,
          wvs, bvs, wscr, bscr, sem):
    i = pl.program_id(0)

    @pl.when(i == 0)
    def _prologue():
        cps = []
        for k in range(_B):
            c8 = pl.multiple_of((cam_s[k] // 8) * 8, 8)
            l0 = pl.multiple_of((id_s[k] // 128) * 128, 128)
            cps.append(pltpu.make_async_copy(
                wt_any.at[:, pl.ds(c8, 8), pl.ds(l0, 128)], wscr.at[k], sem))
            cps.append(pltpu.make_async_copy(
                bt_any.at[:, pl.ds(c8, 8), pl.ds(l0, 128)], bscr.at[k], sem))
        for cp in cps:
            cp.start()
        for cp in cps:
            cp.wait()
        sub_i = jax.lax.broadcasted_iota(jnp.int32, (8, 128), 0)
        lane_i = jax.lax.broadcasted_iota(jnp.int32, (8, 128), 1)
        lane3 = jax.lax.broadcasted_iota(jnp.int32, (1, 3), 1)
        for k in range(_B):
            ci = cam_s[k]
            ii = id_s[k]
            msk = jnp.where((sub_i == ci % 8) & (lane_i == ii % 128), 1.0, 0.0)
            wcrow = wcam_ref[pl.ds(ci, 1), :] + wident_ref[pl.ds(ii, 1), :]
            bcrow = bcam_ref[pl.ds(ci, 1), :] + bident_ref[pl.ds(ii, 1), :]
            for c in range(3):
                one = jnp.where(lane3 == c, 1.0, 0.0)
                wvs[k, c] = (jnp.sum(wcrow * one)
                             + 10.0 * jnp.sum(wscr[k, c] * msk))
                bvs[k, c] = (jnp.sum(bcrow * one)
                             + 10.0 * jnp.sum(bscr[k, c] * msk))

    for j in range(_G):
        r = i * _G + j
        b = r // 3
        c = r - 3 * b
        out_ref[j] = img_ref[j] * wvs[b, c] + bvs[b, c]


def kernel(image, camindex, idindex, wcam, bcam, wident, bident, w, b):
    bsz, ch, h, ww = image.shape
    cam = camindex.astype(jnp.int32)
    idn = idindex.astype(jnp.int32)
    wt = jnp.transpose(w, (2, 0, 1))
    bt = jnp.transpose(b, (2, 0, 1))
    grid_spec = pltpu.PrefetchScalarGridSpec(
        num_scalar_prefetch=2,
        grid=(_ROWS // _G,),
        in_specs=[
            pl.BlockSpec((100, 3), lambda i, cs, ids: (0, 0)),
            pl.BlockSpec((100, 3), lambda i, cs, ids: (0, 0)),
            pl.BlockSpec((5000, 3), lambda i, cs, ids: (0, 0)),
            pl.BlockSpec((5000, 3), lambda i, cs, ids: (0, 0)),
            pl.BlockSpec(memory_space=pl.ANY),
            pl.BlockSpec(memory_space=pl.ANY),
            pl.BlockSpec((_G, h, ww), lambda i, cs, ids: (i, 0, 0)),
        ],
        out_specs=pl.BlockSpec((_G, h, ww), lambda i, cs, ids: (i, 0, 0)),
        scratch_shapes=[
            pltpu.SMEM((_B, 3), jnp.float32),
            pltpu.SMEM((_B, 3), jnp.float32),
            pltpu.VMEM((_B, 3, 8, 128), jnp.float32),
            pltpu.VMEM((_B, 3, 8, 128), jnp.float32),
            pltpu.SemaphoreType.DMA,
        ],
    )
    out = pl.pallas_call(
        _body,
        grid_spec=grid_spec,
        out_shape=jax.ShapeDtypeStruct((_ROWS, h, ww), jnp.float32),
    )(cam, idn, wcam, bcam, wident, bident, wt, bt,
      image.reshape(bsz * ch, h, ww))
    return out.reshape(bsz, ch, h, ww)
